# baseline (speedup 1.0000x reference)
import functools

import jax
import jax.numpy as jnp
from jax import lax
from jax.experimental import pallas as pl
from jax.experimental.pallas import tpu as pltpu
from jax.experimental.pallas import tpu_sc as plsc

N_NODES = 100000
N_EDGES = 3200000

NC = 2
NS = 16
NW = NC * NS

C = 2000
CHUNKS = 50
EPW = C * CHUNKS

ACC = 100352
ACC_T = ACC // NS
G = C // 16

_MESH = plsc.VectorSubcoreMesh(
    core_axis_name="c", subcore_axis_name="s", num_cores=NC, num_subcores=NS
)

_EDGE_VMEM = (
    [pltpu.VMEM((2 * C,), jnp.int32)] * 2
    + [pltpu.VMEM((C,), jnp.int32)] * 2
    + [pltpu.VMEM((C,), jnp.int32)] * 2
    + [pltpu.VMEM((C,), jnp.int32)] * 2
    + [pltpu.VMEM((C,), jnp.float32)] * 18
)


@functools.partial(
    pl.kernel,
    out_type=jax.ShapeDtypeStruct((NC * ACC,), jnp.float32),
    mesh=_MESH,
    compiler_params=pltpu.CompilerParams(needs_layout_passes=False),
    scratch_types=_EDGE_VMEM
    + [
        pltpu.VMEM((ACC_T,), jnp.float32),
        pltpu.VMEM_SHARED((ACC,), jnp.float32),
        pltpu.VMEM_SHARED((ACC,), jnp.float32),
        pltpu.VMEM_SHARED((ACC,), jnp.float32),
        pltpu.VMEM_SHARED((ACC,), jnp.float32),
        pltpu.SemaphoreType.DMA,
        pltpu.SemaphoreType.DMA,
        pltpu.SemaphoreType.DMA,
    ],
)
def _bond_energy(adj_hbm, len_hbm, par_hbm, x_hbm, y_hbm, z_hbm,
                 out_hbm,
                 adj_a, adj_b, idx0_a, idx0_b, idx1_a, idx1_b, sidx_a, sidx_b,
                 len_a, len_b, par_a, par_b,
                 ax_a, ax_b, ay_a, ay_b, az_a, az_b,
                 bx_a, bx_b, by_a, by_b, bz_a, bz_b, eb_a, eb_b,
                 zbuf, acc_sh, x_sh, y_sh, z_sh, sem_l, sem_g, sem_s):
    adjv = (adj_a, adj_b)
    idx0 = (idx0_a, idx0_b)
    idx1 = (idx1_a, idx1_b)
    sidx = (sidx_a, sidx_b)
    ln = (len_a, len_b)
    pr = (par_a, par_b)
    ax = (ax_a, ax_b)
    ay = (ay_a, ay_b)
    az = (az_a, az_b)
    bx = (bx_a, bx_b)
    by = (by_a, by_b)
    bz = (bz_a, bz_b)
    eb = (eb_a, eb_b)

    cid = lax.axis_index("c")
    sid = lax.axis_index("s")
    wid = cid * NS + sid
    tslice = pl.ds(sid * ACC_T, ACC_T)

    def _zero(i, carry):
        zbuf[pl.ds(i * 16, 16)] = jnp.zeros((16,), jnp.float32)
        return carry

    lax.fori_loop(0, ACC_T // 16, _zero, 0)
    pltpu.sync_copy(zbuf, acc_sh.at[tslice])

    pltpu.sync_copy(x_hbm.at[tslice], zbuf)
    pltpu.sync_copy(zbuf, x_sh.at[tslice])
    pltpu.sync_copy(y_hbm.at[tslice], zbuf)
    pltpu.sync_copy(zbuf, y_sh.at[tslice])
    pltpu.sync_copy(z_hbm.at[tslice], zbuf)
    pltpu.sync_copy(zbuf, z_sh.at[tslice])
    plsc.subcore_barrier()

    base_w = wid * EPW

    def lin_parts(t, b):
        base = base_w + t * C
        sl = pl.ds(base, C)
        return (
            (adj_hbm.at[pl.ds(2 * base, 2 * C)], adjv[b]),
            (len_hbm.at[sl], ln[b]),
            (par_hbm.at[sl], pr[b]),
        )

    iota2 = lax.iota(jnp.int32, 16) * 2

    def deint(b):
        def _d(g, carry):
            ev = g * 32 + iota2
            gs = pl.ds(g * 16, 16)
            idx0[b][gs] = plsc.load_gather(adjv[b], [ev])
            idx1[b][gs] = plsc.load_gather(adjv[b], [ev + 1])
            return carry

        lax.fori_loop(0, G, _d, 0)

    def lin_issue(t, b):
        for src, dst in lin_parts(t, b):
            pltpu.async_copy(src, dst, sem_l)

    def lin_wait(t, b):
        for src, dst in lin_parts(t, b):
            pltpu.make_async_copy(src, dst, sem_l).wait()

    def gat_parts(b):
        return (
            (x_sh.at[idx0[b]], ax[b]),
            (y_sh.at[idx0[b]], ay[b]),
            (z_sh.at[idx0[b]], az[b]),
            (x_sh.at[idx1[b]], bx[b]),
            (y_sh.at[idx1[b]], by[b]),
            (z_sh.at[idx1[b]], bz[b]),
        )

    def gat_issue(b):
        for src, dst in gat_parts(b):
            pltpu.async_copy(src, dst, sem_g)

    def gat_wait(b):
        for src, dst in gat_parts(b):
            pltpu.make_async_copy(src, dst, sem_g).wait()

    def compute(b):
        def _group(g, gcarry):
            gs = pl.ds(g * 16, 16)
            dx = ax[b][gs] - bx[b][gs]
            dy = ay[b][gs] - by[b][gs]
            dz = az[b][gs] - bz[b][gs]
            d2 = dx * dx + dy * dy + dz * dz
            d2c = jnp.maximum(d2, jnp.float32(1e-30))
            bits = lax.bitcast_convert_type(d2c, jnp.int32)
            r = lax.bitcast_convert_type(
                jnp.int32(0x5F3759DF) - (bits >> 1), jnp.float32
            )
            r = r * (1.5 - 0.5 * d2c * r * r)
            r = r * (1.5 - 0.5 * d2c * r * r)
            r = r * (1.5 - 0.5 * d2c * r * r)
            e = d2 * r
            diff = e - ln[b][gs]
            eb[b][gs] = 0.5 * pr[b][gs] * diff * diff
            sidx[b][gs] = idx0[b][gs]
            return gcarry

        lax.fori_loop(0, G, _group, 0)

    def scat_issue(b):
        pltpu.async_copy(eb[b], acc_sh.at[sidx[b]], sem_s, add=True)

    def scat_wait(b):
        pltpu.make_async_copy(eb[b], acc_sh.at[sidx[b]], sem_s).wait()

    for src, dst in lin_parts(0, 0):
        pltpu.sync_copy(src, dst)
    deint(0)
    gat_issue(0)
    lin_issue(1, 1)

    def _two(u, carry):
        t0 = 2 * u
        gat_wait(0)
        lin_wait(t0 + 1, 1)
        deint(1)
        gat_issue(1)

        @pl.when(u > 0)
        def _():
            scat_wait(0)

        compute(0)
        scat_issue(0)

        @pl.when(u + 1 < CHUNKS // 2)
        def _():
            lin_issue(t0 + 2, 0)

        gat_wait(1)

        @pl.when(u + 1 < CHUNKS // 2)
        def _():
            lin_wait(t0 + 2, 0)
            deint(0)
            gat_issue(0)

        @pl.when(u > 0)
        def _():
            scat_wait(1)

        compute(1)
        scat_issue(1)

        @pl.when(u + 1 < CHUNKS // 2)
        def _():
            lin_issue(t0 + 3, 1)

        return carry

    lax.fori_loop(0, CHUNKS // 2, _two, 0)
    scat_wait(0)
    scat_wait(1)
    plsc.subcore_barrier()

    pltpu.sync_copy(acc_sh.at[tslice], zbuf)
    pltpu.sync_copy(zbuf, out_hbm.at[pl.ds(cid * ACC + sid * ACC_T, ACC_T)])


HALF = ACC_T // 2


@functools.partial(
    pl.kernel,
    out_type=jax.ShapeDtypeStruct((ACC,), jnp.float32),
    mesh=_MESH,
    scratch_types=[
        pltpu.VMEM((HALF,), jnp.float32),
        pltpu.VMEM((HALF,), jnp.float32),
    ],
)
def _combine(p_hbm, out_hbm, a_v, b_v):
    cid = lax.axis_index("c")
    sid = lax.axis_index("s")
    off = sid * ACC_T + cid * HALF
    pltpu.sync_copy(p_hbm.at[pl.ds(off, HALF)], a_v)
    pltpu.sync_copy(p_hbm.at[pl.ds(ACC + off, HALF)], b_v)

    def _add(i, carry):
        a_v[pl.ds(i * 16, 16)] = a_v[pl.ds(i * 16, 16)] + b_v[pl.ds(i * 16, 16)]
        return carry

    lax.fori_loop(0, HALF // 16, _add, 0)
    pltpu.sync_copy(a_v, out_hbm.at[pl.ds(off, HALF)])


def kernel(xyz, bond_adj, bond_len, bond_par):
    adjf = bond_adj.astype(jnp.int32).reshape(2 * N_EDGES)
    ln = bond_len[:, 0]
    pr = bond_par[:, 0]
    npad = ACC - N_NODES
    znf = jnp.zeros((npad,), jnp.float32)
    xp = jnp.concatenate([xyz[:, 0], znf])
    yp = jnp.concatenate([xyz[:, 1], znf])
    zp = jnp.concatenate([xyz[:, 2], znf])
    partials = _bond_energy(adjf, ln, pr, xp, yp, zp)
    out = _combine(partials)
    return out[:N_NODES][:, None]


# speedup vs baseline: 13.1070x; 13.1070x over previous
import functools

import jax
import jax.numpy as jnp
from jax import lax
from jax.experimental import pallas as pl
from jax.experimental.pallas import tpu as pltpu
from jax.experimental.pallas import tpu_sc as plsc

N_NODES = 100000
N_EDGES = 3200000

NC = 2
NS = 16
NW = NC * NS

C = 2000
CHUNKS = 50
EPW = C * CHUNKS

ACC = 100352
ACC_T = ACC // NS
G = C // 16

_MESH = plsc.VectorSubcoreMesh(
    core_axis_name="c", subcore_axis_name="s", num_cores=NC, num_subcores=NS
)

_EDGE_VMEM = (
    [pltpu.VMEM((C,), jnp.int32)] * 2
    + [pltpu.VMEM((C,), jnp.int32)] * 2
    + [pltpu.VMEM((C,), jnp.int32)] * 2
    + [pltpu.VMEM((C,), jnp.float32)] * 18
)


@functools.partial(
    pl.kernel,
    out_type=jax.ShapeDtypeStruct((NC * ACC,), jnp.float32),
    mesh=_MESH,
    scratch_types=_EDGE_VMEM
    + [
        pltpu.VMEM((ACC_T,), jnp.float32),
        pltpu.VMEM_SHARED((ACC,), jnp.float32),
        pltpu.VMEM_SHARED((ACC,), jnp.float32),
        pltpu.VMEM_SHARED((ACC,), jnp.float32),
        pltpu.VMEM_SHARED((ACC,), jnp.float32),
        pltpu.SemaphoreType.DMA,
        pltpu.SemaphoreType.DMA,
        pltpu.SemaphoreType.DMA,
    ],
)
def _bond_energy(idx0_hbm, idx1_hbm, len_hbm, par_hbm, x_hbm, y_hbm, z_hbm,
                 out_hbm,
                 idx0_a, idx0_b, idx1_a, idx1_b, sidx_a, sidx_b,
                 len_a, len_b, par_a, par_b,
                 ax_a, ax_b, ay_a, ay_b, az_a, az_b,
                 bx_a, bx_b, by_a, by_b, bz_a, bz_b, eb_a, eb_b,
                 zbuf, acc_sh, x_sh, y_sh, z_sh, sem_l, sem_g, sem_s):
    idx0 = (idx0_a, idx0_b)
    idx1 = (idx1_a, idx1_b)
    sidx = (sidx_a, sidx_b)
    ln = (len_a, len_b)
    pr = (par_a, par_b)
    ax = (ax_a, ax_b)
    ay = (ay_a, ay_b)
    az = (az_a, az_b)
    bx = (bx_a, bx_b)
    by = (by_a, by_b)
    bz = (bz_a, bz_b)
    eb = (eb_a, eb_b)

    cid = lax.axis_index("c")
    sid = lax.axis_index("s")
    wid = cid * NS + sid
    tslice = pl.ds(sid * ACC_T, ACC_T)

    def _zero(i, carry):
        zbuf[pl.ds(i * 16, 16)] = jnp.zeros((16,), jnp.float32)
        return carry

    lax.fori_loop(0, ACC_T // 16, _zero, 0)
    pltpu.sync_copy(zbuf, acc_sh.at[tslice])

    pltpu.sync_copy(x_hbm.at[tslice], zbuf)
    pltpu.sync_copy(zbuf, x_sh.at[tslice])
    pltpu.sync_copy(y_hbm.at[tslice], zbuf)
    pltpu.sync_copy(zbuf, y_sh.at[tslice])
    pltpu.sync_copy(z_hbm.at[tslice], zbuf)
    pltpu.sync_copy(zbuf, z_sh.at[tslice])
    plsc.subcore_barrier()

    base_w = wid * EPW

    def lin_parts(t, b):
        base = base_w + t * C
        sl = pl.ds(base, C)
        return (
            (idx0_hbm.at[sl], idx0[b]),
            (idx1_hbm.at[sl], idx1[b]),
            (len_hbm.at[sl], ln[b]),
            (par_hbm.at[sl], pr[b]),
        )

    def lin_issue(t, b):
        for src, dst in lin_parts(t, b):
            pltpu.async_copy(src, dst, sem_l)

    def lin_wait(t, b):
        for src, dst in lin_parts(t, b):
            pltpu.make_async_copy(src, dst, sem_l).wait()

    def gat_parts(b):
        return (
            (x_sh.at[idx0[b]], ax[b]),
            (y_sh.at[idx0[b]], ay[b]),
            (z_sh.at[idx0[b]], az[b]),
            (x_sh.at[idx1[b]], bx[b]),
            (y_sh.at[idx1[b]], by[b]),
            (z_sh.at[idx1[b]], bz[b]),
        )

    def gat_issue(b):
        for src, dst in gat_parts(b):
            pltpu.async_copy(src, dst, sem_g)

    def gat_wait(b):
        for src, dst in gat_parts(b):
            pltpu.make_async_copy(src, dst, sem_g).wait()

    def compute(b):
        def _group(g, gcarry):
            gs = pl.ds(g * 16, 16)
            dx = ax[b][gs] - bx[b][gs]
            dy = ay[b][gs] - by[b][gs]
            dz = az[b][gs] - bz[b][gs]
            d2 = dx * dx + dy * dy + dz * dz
            d2c = jnp.maximum(d2, jnp.float32(1e-30))
            bits = lax.bitcast_convert_type(d2c, jnp.int32)
            r = lax.bitcast_convert_type(
                jnp.int32(0x5F3759DF) - (bits >> 1), jnp.float32
            )
            r = r * (1.5 - 0.5 * d2c * r * r)
            r = r * (1.5 - 0.5 * d2c * r * r)
            r = r * (1.5 - 0.5 * d2c * r * r)
            e = d2 * r
            diff = e - ln[b][gs]
            eb[b][gs] = 0.5 * pr[b][gs] * diff * diff
            sidx[b][gs] = idx0[b][gs]
            return gcarry

        lax.fori_loop(0, G, _group, 0)

    def scat_issue(b):
        pltpu.async_copy(eb[b], acc_sh.at[sidx[b]], sem_s, add=True)

    def scat_wait(b):
        pltpu.make_async_copy(eb[b], acc_sh.at[sidx[b]], sem_s).wait()

    for src, dst in lin_parts(0, 0):
        pltpu.sync_copy(src, dst)
    gat_issue(0)
    lin_issue(1, 1)

    def _two(u, carry):
        t0 = 2 * u
        gat_wait(0)
        lin_wait(t0 + 1, 1)
        gat_issue(1)

        @pl.when(u > 0)
        def _():
            scat_wait(0)

        compute(0)
        scat_issue(0)

        @pl.when(u + 1 < CHUNKS // 2)
        def _():
            lin_issue(t0 + 2, 0)

        gat_wait(1)

        @pl.when(u + 1 < CHUNKS // 2)
        def _():
            lin_wait(t0 + 2, 0)
            gat_issue(0)

        @pl.when(u > 0)
        def _():
            scat_wait(1)

        compute(1)
        scat_issue(1)

        @pl.when(u + 1 < CHUNKS // 2)
        def _():
            lin_issue(t0 + 3, 1)

        return carry

    lax.fori_loop(0, CHUNKS // 2, _two, 0)
    scat_wait(0)
    scat_wait(1)
    plsc.subcore_barrier()

    pltpu.sync_copy(acc_sh.at[tslice], zbuf)
    pltpu.sync_copy(zbuf, out_hbm.at[pl.ds(cid * ACC + sid * ACC_T, ACC_T)])


HALF = ACC_T // 2


@functools.partial(
    pl.kernel,
    out_type=jax.ShapeDtypeStruct((ACC,), jnp.float32),
    mesh=_MESH,
    scratch_types=[
        pltpu.VMEM((HALF,), jnp.float32),
        pltpu.VMEM((HALF,), jnp.float32),
    ],
)
def _combine(p_hbm, out_hbm, a_v, b_v):
    cid = lax.axis_index("c")
    sid = lax.axis_index("s")
    off = sid * ACC_T + cid * HALF
    pltpu.sync_copy(p_hbm.at[pl.ds(off, HALF)], a_v)
    pltpu.sync_copy(p_hbm.at[pl.ds(ACC + off, HALF)], b_v)

    def _add(i, carry):
        a_v[pl.ds(i * 16, 16)] = a_v[pl.ds(i * 16, 16)] + b_v[pl.ds(i * 16, 16)]
        return carry

    lax.fori_loop(0, HALF // 16, _add, 0)
    pltpu.sync_copy(a_v, out_hbm.at[pl.ds(off, HALF)])


def kernel(xyz, bond_adj, bond_len, bond_par):
    idx0 = bond_adj[:, 0].astype(jnp.int32)
    idx1 = bond_adj[:, 1].astype(jnp.int32)
    ln = bond_len[:, 0]
    pr = bond_par[:, 0]
    npad = ACC - N_NODES
    znf = jnp.zeros((npad,), jnp.float32)
    xp = jnp.concatenate([xyz[:, 0], znf])
    yp = jnp.concatenate([xyz[:, 1], znf])
    zp = jnp.concatenate([xyz[:, 2], znf])
    partials = _bond_energy(idx0, idx1, ln, pr, xp, yp, zp)
    out = _combine(partials)
    return out[:N_NODES][:, None]
